# baseline (speedup 1.0000x reference)
import functools

import jax
import jax.numpy as jnp
from jax import lax
from jax.experimental import pallas as pl
from jax.experimental.pallas import tpu as pltpu
from jax.experimental.pallas import tpu_sc as plsc

D = 64
S = 200
NW = 32
CHUNK = 128
LANES = 16


def _sc_body(ids_hbm, tok_hbm, pos_hbm, out_hbm,
             idx_v, pos_v, buf0, buf1, sg0, sg1, ss0, ss1):
    cpw = ids_hbm.shape[0] // NW
    wid = lax.axis_index("s") * 2 + lax.axis_index("c")
    row0 = wid * (cpw * CHUNK)
    irow0 = wid * cpw

    pltpu.sync_copy(ids_hbm.at[pl.ds(irow0, cpw)], idx_v)
    pltpu.sync_copy(pos_hbm.at[pl.ds(0, S)], pos_v.at[pl.ds(0, S)])
    pltpu.sync_copy(pos_hbm.at[pl.ds(0, S)], pos_v.at[pl.ds(S, S)])

    bufs = (buf0, buf1)
    gsems = (sg0, sg1)
    ssems = (ss0, ss1)

    def start_gather(c, b):
        pltpu.async_copy(tok_hbm.at[idx_v.at[c]], bufs[b], gsems[b])

    def wait_gather(b):
        pltpu.make_async_copy(tok_hbm.at[pl.ds(0, CHUNK)], bufs[b],
                              gsems[b]).wait()

    def start_scatter(c, b):
        pltpu.async_copy(bufs[b], out_hbm.at[pl.ds(row0 + c * CHUNK, CHUNK)],
                         ssems[b])

    def wait_scatter(b):
        pltpu.make_async_copy(bufs[b], out_hbm.at[pl.ds(0, CHUNK)],
                              ssems[b]).wait()

    def add_pos(c, b):
        s0 = lax.rem(c * CHUNK, S)

        def row_body(r, carry):
            for dr in range(4):
                rr = r * 4 + dr
                pr = s0 + rr
                for col in range(D // LANES):
                    x = pos_v[pr, pl.ds(col * LANES, LANES)]
                    plsc.addupdate(bufs[b].at[rr, pl.ds(col * LANES, LANES)], x)
            return carry

        lax.fori_loop(0, CHUNK // 4, row_body, 0)

    start_gather(0, 0)

    def outer(i, carry):
        for b in range(2):
            c = i * 2 + b
            nb = 1 - b

            @pl.when(c + 1 < cpw)
            def _():
                @pl.when(c >= 1)
                def _():
                    wait_scatter(nb)
                start_gather(c + 1, nb)

            wait_gather(b)
            add_pos(c, b)
            start_scatter(c, b)
        return carry

    lax.fori_loop(0, cpw // 2, outer, 0)
    wait_scatter(0)
    wait_scatter(1)


def kernel(input_ids, tok_table, pos_table):
    bsz, seq_len = input_ids.shape
    rows = bsz * seq_len
    ids2 = input_ids.reshape(rows // CHUNK, CHUNK)

    mesh = plsc.VectorSubcoreMesh(core_axis_name="c", subcore_axis_name="s")
    cpw = ids2.shape[0] // NW
    run = pl.kernel(
        _sc_body, mesh=mesh,
        out_type=jax.ShapeDtypeStruct((rows, D), jnp.float32),
        scratch_types=[
            pltpu.VMEM((cpw, CHUNK), jnp.int32),
            pltpu.VMEM((2 * S, D), jnp.float32),
            pltpu.VMEM((CHUNK, D), jnp.float32),
            pltpu.VMEM((CHUNK, D), jnp.float32),
            pltpu.SemaphoreType.DMA,
            pltpu.SemaphoreType.DMA,
            pltpu.SemaphoreType.DMA,
            pltpu.SemaphoreType.DMA,
        ],
        compiler_params=pltpu.CompilerParams(use_tc_tiling_on_sc=False),
    )
    out = run(ids2, tok_table, pos_table)
    return out.reshape(bsz, seq_len, D)
